# Initial kernel scaffold; baseline (speedup 1.0000x reference)
import jax
import jax.numpy as jnp
from jax.experimental import pallas as pl


def kernel(inputs):
    raise NotImplementedError("write your pallas kernel here")


# speedup vs baseline: 4.7019x; 4.7019x over previous
import functools

import jax
import jax.numpy as jnp
from jax import lax
from jax.experimental import pallas as pl
from jax.experimental.pallas import tpu as pltpu
from jax.experimental.pallas import tpu_sc as plsc

B = 16
N = 16384
S = 1024
L = 16
NC, NS = 2, 16
CHUNKS = N // L

_MESH = plsc.VectorSubcoreMesh(
    core_axis_name="c", subcore_axis_name="s", num_cores=NC, num_subcores=NS
)


def _fps_body(x_hbm, y_hbm, z_hbm, out_hbm, xv, yv, zv, dist_v, out_v):
    wid = lax.axis_index("s") * NC + lax.axis_index("c")

    @pl.when(wid < B)
    def _():
        b = wid
        pltpu.sync_copy(x_hbm.at[b], xv)
        pltpu.sync_copy(y_hbm.at[b], yv)
        pltpu.sync_copy(z_hbm.at[b], zv)

        big = jnp.full((L,), 1e10, jnp.float32)

        @pl.loop(0, CHUNKS)
        def _init(j):
            dist_v[pl.ds(j * L, L)] = big

        lane = lax.iota(jnp.int32, L)
        m0 = lane == 0

        def outer(t, far):
            far_vec = jnp.full((L,), far, jnp.int32)
            cx = plsc.load_gather(xv, [far_vec])
            cy = plsc.load_gather(yv, [far_vec])
            cz = plsc.load_gather(zv, [far_vec])

            pos = jnp.full((L,), 3 * t, jnp.int32)
            plsc.store_scatter(out_v, [pos], cx, mask=m0)
            plsc.store_scatter(out_v, [pos + 1], cy, mask=m0)
            plsc.store_scatter(out_v, [pos + 2], cz, mask=m0)

            def inner(j, carry):
                best, bidx = carry
                sl = pl.ds(j * L, L)
                dx = xv[sl] - cx
                dy = yv[sl] - cy
                dz = zv[sl] - cz
                d = dx * dx + dy * dy + dz * dz
                nd = jnp.minimum(dist_v[sl], d)
                dist_v[sl] = nd
                better = nd > best
                best = jnp.where(better, nd, best)
                bidx = jnp.where(better, lane + j * L, bidx)
                return best, bidx

            best0 = jnp.full((L,), -1.0, jnp.float32)
            bidx0 = jnp.zeros((L,), jnp.int32)
            best, bidx = lax.fori_loop(0, CHUNKS, inner, (best0, bidx0))

            mx = jnp.max(best)
            cand = jnp.where(best == mx, bidx, jnp.int32(2**31 - 1))
            return jnp.min(cand)

        lax.fori_loop(0, S, outer, jnp.int32(0))
        pltpu.sync_copy(out_v, out_hbm.at[b])


_fps = pl.kernel(
    _fps_body,
    out_type=jax.ShapeDtypeStruct((B, 3 * S), jnp.float32),
    mesh=_MESH,
    compiler_params=pltpu.CompilerParams(needs_layout_passes=False),
    scratch_types=[
        pltpu.VMEM((N,), jnp.float32),
        pltpu.VMEM((N,), jnp.float32),
        pltpu.VMEM((N,), jnp.float32),
        pltpu.VMEM((N,), jnp.float32),
        pltpu.VMEM((3 * S,), jnp.float32),
    ],
)


def kernel(inputs):
    x = inputs[:, :, 0]
    y = inputs[:, :, 1]
    z = inputs[:, :, 2]
    out = _fps(x, y, z)
    return out.reshape(B, S, 3)


# speedup vs baseline: 18.4063x; 3.9147x over previous
import functools

import jax
import jax.numpy as jnp
from jax import lax
from jax.experimental import pallas as pl
from jax.experimental.pallas import tpu as pltpu
from jax.experimental.pallas import tpu_sc as plsc

B = 16
N = 16384
S = 1024
L = 16
NC, NS = 2, 16
CHUNKS = N // L

_MESH = plsc.VectorSubcoreMesh(
    core_axis_name="c", subcore_axis_name="s", num_cores=NC, num_subcores=NS
)


def _fps_body(x_hbm, y_hbm, z_hbm, out_hbm, xv, yv, zv, dist_v, out_v):
    wid = lax.axis_index("s") * NC + lax.axis_index("c")

    @pl.when(wid < B)
    def _():
        b = wid
        pltpu.sync_copy(x_hbm.at[b], xv)
        pltpu.sync_copy(y_hbm.at[b], yv)
        pltpu.sync_copy(z_hbm.at[b], zv)

        big = jnp.full((L,), 1e10, jnp.float32)

        @pl.loop(0, CHUNKS)
        def _init(j):
            dist_v[pl.ds(j * L, L)] = big

        lane = lax.iota(jnp.int32, L)
        m0 = lane == 0

        def outer(t, far):
            far_vec = jnp.full((L,), far, jnp.int32)
            cx = plsc.load_gather(xv, [far_vec])
            cy = plsc.load_gather(yv, [far_vec])
            cz = plsc.load_gather(zv, [far_vec])

            pos = jnp.full((L,), 3 * t, jnp.int32)
            plsc.store_scatter(out_v, [pos], cx, mask=m0)
            plsc.store_scatter(out_v, [pos + 1], cy, mask=m0)
            plsc.store_scatter(out_v, [pos + 2], cz, mask=m0)

            best0 = jnp.full((L,), -1.0, jnp.float32)
            bchunk0 = jnp.zeros((L,), jnp.int32)

            @plsc.parallel_loop(0, CHUNKS, unroll=8, carry=(best0, bchunk0))
            def inner(j, carry):
                best, bchunk = carry
                sl = pl.ds(j * L, L)
                dx = xv[sl] - cx
                dy = yv[sl] - cy
                dz = zv[sl] - cz
                d = dx * dx + dy * dy + dz * dz
                nd = jnp.minimum(dist_v[sl], d)
                dist_v[sl] = nd
                better = nd > best
                best = jnp.maximum(best, nd)
                bchunk = jnp.where(better, jnp.full((L,), j, jnp.int32), bchunk)
                return best, bchunk

            best, bchunk = inner

            bidx = bchunk * L + lane
            mx = jnp.max(best)
            cand = jnp.where(best == mx, bidx, jnp.int32(2**31 - 1))
            return jnp.min(cand)

        lax.fori_loop(0, S, outer, jnp.int32(0))
        pltpu.sync_copy(out_v, out_hbm.at[b])


_fps = pl.kernel(
    _fps_body,
    out_type=jax.ShapeDtypeStruct((B, 3 * S), jnp.float32),
    mesh=_MESH,
    compiler_params=pltpu.CompilerParams(needs_layout_passes=False),
    scratch_types=[
        pltpu.VMEM((N,), jnp.float32),
        pltpu.VMEM((N,), jnp.float32),
        pltpu.VMEM((N,), jnp.float32),
        pltpu.VMEM((N,), jnp.float32),
        pltpu.VMEM((3 * S,), jnp.float32),
    ],
)


def kernel(inputs):
    x = inputs[:, :, 0]
    y = inputs[:, :, 1]
    z = inputs[:, :, 2]
    out = _fps(x, y, z)
    return out.reshape(B, S, 3)
